# Initial kernel scaffold; baseline (speedup 1.0000x reference)
import jax
import jax.numpy as jnp
from jax.experimental import pallas as pl


def kernel(x, edge_index, W_enc, b_enc, W1, b1, W2, b2, eps):
    raise NotImplementedError("write your pallas kernel here")


# speedup vs baseline: 4.8864x; 4.8864x over previous
import functools

import jax
import jax.numpy as jnp
from jax import lax
from jax.experimental import pallas as pl
from jax.experimental.pallas import tpu as pltpu
from jax.experimental.pallas import tpu_sc as plsc

N = 10000
E = 320000
D = 128
NLAYERS = 3

NC = 2
NS = 16
NW = NC * NS
EPW = E // NW
K = 80
ITERS = EPW // K
NSUB_IO = 10
ROWS_PER_IO = N // NSUB_IO
ZROWS = 200


def _sc_agg_body(h_hbm, src_hbm, dst_hbm, out_hbm,
                 src_v, dst_v, rows_v, zbuf_v, acc_sh, sem):
    cid = lax.axis_index("c")
    sid = lax.axis_index("s")
    wid = sid * NC + cid

    zv = jnp.zeros((16,), jnp.float32)

    def zero_row(r, _):
        for c8 in range(D // 16):
            zbuf_v[r, pl.ds(c8 * 16, 16)] = zv
        return 0

    lax.fori_loop(0, ZROWS, zero_row, 0)

    @pl.when(sid < NSUB_IO)
    def _zero_acc():
        for j in range(ROWS_PER_IO // ZROWS):
            pltpu.sync_copy(
                zbuf_v, acc_sh.at[pl.ds(sid * ROWS_PER_IO + j * ZROWS, ZROWS)])

    plsc.subcore_barrier()

    def body(i, _):
        base = wid * EPW + i * K
        pltpu.sync_copy(src_hbm.at[pl.ds(base, K)], src_v)
        pltpu.sync_copy(dst_hbm.at[pl.ds(base, K)], dst_v)
        pltpu.async_copy(h_hbm.at[src_v], rows_v, sem).wait()
        pltpu.sync_copy(rows_v, acc_sh.at[dst_v], add=True)
        return 0

    lax.fori_loop(0, ITERS, body, 0)
    plsc.subcore_barrier()

    @pl.when(sid < NSUB_IO)
    def _copy_out():
        pltpu.sync_copy(
            acc_sh.at[pl.ds(sid * ROWS_PER_IO, ROWS_PER_IO)],
            out_hbm.at[pl.ds(cid * N + sid * ROWS_PER_IO, ROWS_PER_IO)],
        )


def _sc_agg(h, src, dst):
    mesh = plsc.VectorSubcoreMesh(core_axis_name="c", subcore_axis_name="s")
    return pl.kernel(
        _sc_agg_body,
        out_type=jax.ShapeDtypeStruct((2 * N, D), jnp.float32),
        mesh=mesh,
        scratch_types=[
            pltpu.VMEM((K,), jnp.int32),
            pltpu.VMEM((K,), jnp.int32),
            pltpu.VMEM((K, D), jnp.float32),
            pltpu.VMEM((ZROWS, D), jnp.float32),
            pltpu.VMEM_SHARED((N, D), jnp.float32),
            pltpu.SemaphoreType.DMA,
        ],
    )(h, src, dst)


BM = 1000


def _enc_body(x_ref, w_ref, b_ref, o_ref):
    o_ref[...] = jnp.maximum(
        jnp.dot(x_ref[...], w_ref[...], preferred_element_type=jnp.float32)
        + b_ref[...], 0.0)


def _tc_encoder(x, W_enc, b_enc):
    return pl.pallas_call(
        _enc_body,
        grid=(N // BM,),
        in_specs=[
            pl.BlockSpec((BM, D), lambda i: (i, 0)),
            pl.BlockSpec((D, D), lambda i: (0, 0)),
            pl.BlockSpec((1, D), lambda i: (0, 0)),
        ],
        out_specs=pl.BlockSpec((BM, D), lambda i: (i, 0)),
        out_shape=jax.ShapeDtypeStruct((N, D), jnp.float32),
    )(x, W_enc, b_enc.reshape(1, D))


def _mlp_body(eps_ref, h_ref, a0_ref, a1_ref, w1_ref, b1_ref, w2_ref, b2_ref,
              o_ref):
    scale = 1.0 + eps_ref[0]
    z = scale * h_ref[...] + a0_ref[...] + a1_ref[...]
    z = jnp.maximum(
        jnp.dot(z, w1_ref[...], preferred_element_type=jnp.float32)
        + b1_ref[...], 0.0)
    o_ref[...] = jnp.maximum(
        jnp.dot(z, w2_ref[...], preferred_element_type=jnp.float32)
        + b2_ref[...], 0.0)


def _tc_mlp(h, agg2, W1l, b1l, W2l, b2l, epsl):
    a0 = agg2[:N]
    a1 = agg2[N:]
    return pl.pallas_call(
        _mlp_body,
        grid=(N // BM,),
        in_specs=[
            pl.BlockSpec(memory_space=pltpu.SMEM),
            pl.BlockSpec((BM, D), lambda i: (i, 0)),
            pl.BlockSpec((BM, D), lambda i: (i, 0)),
            pl.BlockSpec((BM, D), lambda i: (i, 0)),
            pl.BlockSpec((D, D), lambda i: (0, 0)),
            pl.BlockSpec((1, D), lambda i: (0, 0)),
            pl.BlockSpec((D, D), lambda i: (0, 0)),
            pl.BlockSpec((1, D), lambda i: (0, 0)),
        ],
        out_specs=pl.BlockSpec((BM, D), lambda i: (i, 0)),
        out_shape=jax.ShapeDtypeStruct((N, D), jnp.float32),
    )(epsl.reshape(1), h, a0, a1, W1l, b1l.reshape(1, D), W2l,
      b2l.reshape(1, D))


def kernel(x, edge_index, W_enc, b_enc, W1, b1, W2, b2, eps):
    src = edge_index[0].astype(jnp.int32)
    dst = edge_index[1].astype(jnp.int32)
    h = _tc_encoder(x, W_enc, b_enc)
    for l in range(NLAYERS):
        agg2 = _sc_agg(h, src, dst)
        h = _tc_mlp(h, agg2, W1[l], b1[l], W2[l], b2[l], eps[l])
    return h


# speedup vs baseline: 11.1863x; 2.2893x over previous
import functools

import jax
import jax.numpy as jnp
from jax import lax
from jax.experimental import pallas as pl
from jax.experimental.pallas import tpu as pltpu
from jax.experimental.pallas import tpu_sc as plsc

N = 10000
E = 320000
D = 128
NLAYERS = 3

NC = 2
NS = 16
NW = NC * NS
EPW = E // NW
K = 80
ITERS = EPW // K
NSUB_IO = 10
ROWS_PER_IO = N // NSUB_IO


def _sc_agg_body(h_hbm, src_hbm, dst_hbm, out_hbm,
                 src_v, dst_v, rows_v, acc_sh, sem, isem):
    cid = lax.axis_index("c")
    sid = lax.axis_index("s")
    wid = sid * NC + cid

    idx_cp0 = pltpu.async_copy(src_hbm.at[wid], src_v, isem)
    idx_cp1 = pltpu.async_copy(dst_hbm.at[wid], dst_v, isem)

    zv = jnp.zeros((16,), jnp.float32)

    def zero_row(r, _):
        for c8 in range(D // 16):
            rows_v[0, r, pl.ds(c8 * 16, 16)] = zv
        return 0

    lax.fori_loop(0, K, zero_row, 0)

    @pl.when(sid < NSUB_IO)
    def _zero_acc():
        for j in range(ROWS_PER_IO // K):
            pltpu.sync_copy(
                rows_v.at[0], acc_sh.at[pl.ds(sid * ROWS_PER_IO + j * K, K)])
        rem = ROWS_PER_IO % K
        if rem:
            pltpu.sync_copy(
                rows_v.at[0, pl.ds(0, rem)],
                acc_sh.at[pl.ds(sid * ROWS_PER_IO + (ROWS_PER_IO // K) * K,
                                rem)])

    idx_cp0.wait()
    idx_cp1.wait()
    plsc.subcore_barrier()

    def fire(i, b):
        return pltpu.async_copy(
            h_hbm.at[src_v.at[pl.ds(i * K, K)]], rows_v.at[b], sem)

    def wait_fire(i, b):
        pltpu.make_async_copy(
            h_hbm.at[src_v.at[pl.ds(i * K, K)]], rows_v.at[b], sem).wait()

    def scatter(i, b):
        pltpu.sync_copy(rows_v.at[b], acc_sh.at[dst_v.at[i]], add=True)

    fire(0, 0)

    def body(j, _):
        i0 = 2 * j
        fire(i0 + 1, 1)
        wait_fire(i0, 0)
        scatter(i0, 0)
        fire(i0 + 2, 0)
        wait_fire(i0 + 1, 1)
        scatter(i0 + 1, 1)
        return 0

    lax.fori_loop(0, (ITERS - 1) // 2, body, 0)
    wait_fire(ITERS - 1, 0)
    scatter(ITERS - 1, 0)
    plsc.subcore_barrier()

    @pl.when(sid < NSUB_IO)
    def _copy_out():
        pltpu.sync_copy(
            acc_sh.at[pl.ds(sid * ROWS_PER_IO, ROWS_PER_IO)],
            out_hbm.at[pl.ds(cid * N + sid * ROWS_PER_IO, ROWS_PER_IO)],
        )


def _sc_agg(h, src3, dst3):
    mesh = plsc.VectorSubcoreMesh(core_axis_name="c", subcore_axis_name="s")
    return pl.kernel(
        _sc_agg_body,
        out_type=jax.ShapeDtypeStruct((2 * N, D), jnp.float32),
        mesh=mesh,
        scratch_types=[
            pltpu.VMEM((EPW,), jnp.int32),
            pltpu.VMEM((ITERS, K), jnp.int32),
            pltpu.VMEM((2, K, D), jnp.float32),
            pltpu.VMEM_SHARED((N, D), jnp.float32),
            pltpu.SemaphoreType.DMA,
            pltpu.SemaphoreType.DMA,
        ],
    )(h, src3, dst3)


BM = 1000


def _enc_body(x_ref, w_ref, b_ref, o_ref):
    o_ref[...] = jnp.maximum(
        jnp.dot(x_ref[...], w_ref[...], preferred_element_type=jnp.float32)
        + b_ref[...], 0.0)


def _tc_encoder(x, W_enc, b_enc):
    return pl.pallas_call(
        _enc_body,
        grid=(N // BM,),
        in_specs=[
            pl.BlockSpec((BM, D), lambda i: (i, 0)),
            pl.BlockSpec((D, D), lambda i: (0, 0)),
            pl.BlockSpec((1, D), lambda i: (0, 0)),
        ],
        out_specs=pl.BlockSpec((BM, D), lambda i: (i, 0)),
        out_shape=jax.ShapeDtypeStruct((N, D), jnp.float32),
    )(x, W_enc, b_enc.reshape(1, D))


def _mlp_body(eps_ref, h_ref, a0_ref, a1_ref, w1_ref, b1_ref, w2_ref, b2_ref,
              o_ref):
    scale = 1.0 + eps_ref[0]
    z = scale * h_ref[...] + a0_ref[...] + a1_ref[...]
    z = jnp.maximum(
        jnp.dot(z, w1_ref[...], preferred_element_type=jnp.float32)
        + b1_ref[...], 0.0)
    o_ref[...] = jnp.maximum(
        jnp.dot(z, w2_ref[...], preferred_element_type=jnp.float32)
        + b2_ref[...], 0.0)


def _tc_mlp(h, agg2, W1l, b1l, W2l, b2l, epsl):
    a0 = agg2[:N]
    a1 = agg2[N:]
    return pl.pallas_call(
        _mlp_body,
        grid=(N // BM,),
        in_specs=[
            pl.BlockSpec(memory_space=pltpu.SMEM),
            pl.BlockSpec((BM, D), lambda i: (i, 0)),
            pl.BlockSpec((BM, D), lambda i: (i, 0)),
            pl.BlockSpec((BM, D), lambda i: (i, 0)),
            pl.BlockSpec((D, D), lambda i: (0, 0)),
            pl.BlockSpec((1, D), lambda i: (0, 0)),
            pl.BlockSpec((D, D), lambda i: (0, 0)),
            pl.BlockSpec((1, D), lambda i: (0, 0)),
        ],
        out_specs=pl.BlockSpec((BM, D), lambda i: (i, 0)),
        out_shape=jax.ShapeDtypeStruct((N, D), jnp.float32),
    )(epsl.reshape(1), h, a0, a1, W1l, b1l.reshape(1, D), W2l,
      b2l.reshape(1, D))


def kernel(x, edge_index, W_enc, b_enc, W1, b1, W2, b2, eps):
    src3 = edge_index[0].astype(jnp.int32).reshape(NW, EPW)
    dst3 = edge_index[1].astype(jnp.int32).reshape(NW, ITERS, K)
    h = _tc_encoder(x, W_enc, b_enc)
    for l in range(NLAYERS):
        agg2 = _sc_agg(h, src3, dst3)
        h = _tc_mlp(h, agg2, W1[l], b1[l], W2[l], b2[l], eps[l])
    return h
